# baseline (speedup 1.0000x reference)
import jax
import jax.numpy as jnp
from jax import lax
from jax.experimental import pallas as pl
from jax.experimental.pallas import tpu as pltpu


_BATCH_TILE = 256


def _gather_kernel(cls_smem, emb_ref, o_ref):
    tb = o_ref.shape[0]
    base = pl.program_id(0) * tb
    for mi in range(tb):
        idx = cls_smem[base + mi]
        v = emb_ref[idx, 0]
        lo = lax.bitcast_convert_type(v << 16, jnp.float32)
        hi = lax.bitcast_convert_type(v & jnp.int32(-65536), jnp.float32)
        o_ref[mi, 0] = lo
        o_ref[mi, 1] = hi


def kernel(cls, cls_emb):
    cls_shape = cls.shape
    batch = 1
    for d in cls_shape:
        batch *= d
    n_class, cond_dim = cls_emb.shape
    out_dtype = cls_emb.dtype

    half = (cond_dim + 1) // 2
    emb_w = cls_emb if 2 * half == cond_dim else jnp.pad(
        cls_emb, ((0, 0), (0, 2 * half - cond_dim)))

    pair = jnp.stack(
        [emb_w[:, :half].astype(jnp.bfloat16),
         emb_w[:, half:].astype(jnp.bfloat16)], axis=-1)
    emb_packed = lax.bitcast_convert_type(pair, jnp.int32)
    emb3 = emb_packed.reshape(n_class, 1, half)

    cls_i32 = jnp.clip(cls.reshape(batch).astype(jnp.int32), 0, n_class - 1)

    tb = min(_BATCH_TILE, batch)
    padded_batch = ((batch + tb - 1) // tb) * tb
    if padded_batch != batch:
        cls_i32 = jnp.pad(cls_i32, (0, padded_batch - batch))

    table_bytes = n_class * half * 4
    vmem_limit = min(
        table_bytes + 4 * tb * 2 * half * 4 + 4 * 1024 * 1024,
        64 * 1024 * 1024,
    )

    out = pl.pallas_call(
        _gather_kernel,
        out_shape=jax.ShapeDtypeStruct((padded_batch, 2, half), jnp.float32),
        grid_spec=pltpu.PrefetchScalarGridSpec(
            num_scalar_prefetch=1,
            grid=(padded_batch // tb,),
            in_specs=[
                pl.BlockSpec((n_class, 1, half), lambda i, c: (0, 0, 0),
                             pipeline_mode=pl.Buffered(1)),
            ],
            out_specs=pl.BlockSpec((tb, 2, half), lambda i, c: (i, 0, 0)),
        ),
        compiler_params=pltpu.CompilerParams(
            dimension_semantics=("parallel",),
            vmem_limit_bytes=int(vmem_limit)),
    )(cls_i32, emb3)

    out = out.reshape(padded_batch, 2 * half)
    if padded_batch != batch or 2 * half != cond_dim:
        out = out[:batch, :cond_dim]
    return out.astype(out_dtype).reshape(*cls_shape, cond_dim)


# speedup vs baseline: 3.0042x; 3.0042x over previous
import jax
import jax.numpy as jnp
from jax.experimental import pallas as pl
from jax.experimental.pallas import tpu as pltpu


_ROWS_PER_STEP = 256


def _dma_gather_kernel(cls_smem, emb_hbm, o_ref, sem):
    rows = o_ref.shape[0]
    base = pl.program_id(0) * rows
    for r in range(rows):
        idx = cls_smem[base + r]
        pltpu.make_async_copy(emb_hbm.at[idx], o_ref.at[r], sem).start()
    pltpu.make_async_copy(emb_hbm.at[pl.ds(0, rows)], o_ref, sem).wait()


def kernel(cls, cls_emb):
    cls_shape = cls.shape
    batch = 1
    for d in cls_shape:
        batch *= d
    n_class, cond_dim = cls_emb.shape
    out_dtype = cls_emb.dtype

    cls_i32 = jnp.clip(cls.reshape(batch).astype(jnp.int32), 0, n_class - 1)

    rows = min(_ROWS_PER_STEP, batch)
    padded_batch = ((batch + rows - 1) // rows) * rows
    if padded_batch != batch:
        cls_i32 = jnp.pad(cls_i32, (0, padded_batch - batch))

    emb3 = cls_emb.reshape(n_class, 1, cond_dim)

    out = pl.pallas_call(
        _dma_gather_kernel,
        out_shape=jax.ShapeDtypeStruct((padded_batch, 1, cond_dim), out_dtype),
        grid_spec=pltpu.PrefetchScalarGridSpec(
            num_scalar_prefetch=1,
            grid=(padded_batch // rows,),
            in_specs=[pl.BlockSpec(memory_space=pl.ANY)],
            out_specs=pl.BlockSpec((rows, 1, cond_dim), lambda i, c: (i, 0, 0)),
            scratch_shapes=[pltpu.SemaphoreType.DMA],
        ),
        compiler_params=pltpu.CompilerParams(
            dimension_semantics=("parallel",),
            vmem_limit_bytes=32 * 1024 * 1024),
    )(cls_i32, emb3)

    if padded_batch != batch:
        out = out[:batch]
    return out.reshape(*cls_shape, cond_dim)
